# baseline (speedup 1.0000x reference)
import functools

import jax
import jax.numpy as jnp
from jax import lax
from jax.experimental import pallas as pl
from jax.experimental.pallas import tpu as pltpu
from jax.experimental.pallas import tpu_sc as plsc

_NC = 2
_NS = 16
_NW = _NC * _NS
_NBUF = 3
_MAIN = 48


@functools.partial(jax.jit, static_argnums=(3,))
def _sc_gather(table, idx_main, idx_tail, b):
    _, d = table.shape
    nw, eps_per_w, _ = idx_main.shape
    l = _MAIN + idx_tail.shape[2]
    nrounds = eps_per_w // _NBUF
    nrem = eps_per_w - nrounds * _NBUF
    mesh = plsc.VectorSubcoreMesh(core_axis_name="c", subcore_axis_name="s")

    @functools.partial(
        pl.kernel,
        mesh=mesh,
        out_type=jax.ShapeDtypeStruct((b, l, d), jnp.float32),
        scratch_types=[
            pltpu.VMEM((eps_per_w, _MAIN), jnp.int32),
            pltpu.VMEM((eps_per_w, 2), jnp.int32),
            [pltpu.VMEM((_MAIN, d), jnp.float32) for _ in range(_NBUF)],
            [pltpu.VMEM((2, d), jnp.float32) for _ in range(_NBUF)],
            [pltpu.SemaphoreType.DMA for _ in range(_NBUF)],
            [pltpu.SemaphoreType.DMA for _ in range(_NBUF)],
        ],
    )
    def k(table_hbm, idxm_hbm, idxt_hbm, out_hbm,
          idxm_v, idxt_v, bufs, tbufs, gsems, wsems):
        wid = lax.axis_index("s") * _NC + lax.axis_index("c")
        base = wid * eps_per_w
        pltpu.sync_copy(idxm_hbm.at[wid], idxm_v)
        pltpu.sync_copy(idxt_hbm.at[wid], idxt_v)

        def start_gather(q, e):
            pltpu.async_copy(table_hbm.at[idxm_v.at[e]], bufs[q], gsems[q])
            pltpu.async_copy(table_hbm.at[idxt_v.at[e]], tbufs[q], gsems[q])

        def wait_gather(q, e):
            pltpu.make_async_copy(
                table_hbm.at[idxm_v.at[e]], bufs[q], gsems[q]).wait()
            pltpu.make_async_copy(
                table_hbm.at[idxt_v.at[e]], tbufs[q], gsems[q]).wait()

        def start_write(q, e):
            pltpu.async_copy(
                bufs[q], out_hbm.at[e, pl.ds(0, _MAIN)], wsems[q])
            pltpu.async_copy(
                tbufs[q], out_hbm.at[e, pl.ds(_MAIN, l - _MAIN)], wsems[q])

        def wait_write(q):
            pltpu.make_async_copy(
                bufs[q], out_hbm.at[base, pl.ds(0, _MAIN)], wsems[q]).wait()
            pltpu.make_async_copy(
                tbufs[q], out_hbm.at[base, pl.ds(_MAIN, l - _MAIN)],
                wsems[q]).wait()

        for q in range(_NBUF):
            start_gather(q, q)

        @pl.loop(0, nrounds)
        def _(t):
            c = t * _NBUF
            for q in range(_NBUF):
                wait_gather(q, c + q)
                start_write(q, base + c + q)

            @pl.when(t + 1 < nrounds)
            def _():
                for q in range(_NBUF):
                    wait_write(q)
                    start_gather(q, c + _NBUF + q)

        for q in range(nrem):
            e = nrounds * _NBUF + q
            wait_write(q)
            start_gather(q, e)
        for q in range(nrem):
            e = nrounds * _NBUF + q
            wait_gather(q, e)
            start_write(q, base + e)

        for q in range(_NBUF):
            wait_write(q)

    return k(table, idx_main, idx_tail)


def kernel(timesteps, table):
    b, l, _ = timesteps.shape
    v = table.shape[0]
    eps_per_w = b // _NW
    idx = timesteps.astype(jnp.int32).reshape(_NW, eps_per_w, l)
    table2 = jnp.concatenate([table, table], axis=0)
    core = (jnp.arange(_NW, dtype=jnp.int32) % _NC)[:, None, None]
    idx = idx + core * v
    return _sc_gather(table2, idx[:, :, :_MAIN], idx[:, :, _MAIN:], b)


# speedup vs baseline: 1.0857x; 1.0857x over previous
import functools

import jax
import jax.numpy as jnp
from jax import lax
from jax.experimental import pallas as pl
from jax.experimental.pallas import tpu as pltpu
from jax.experimental.pallas import tpu_sc as plsc

_NC = 2
_NS = 16
_NW = _NC * _NS
_NBUF = 3
_MAIN = 48


@functools.partial(jax.jit, static_argnums=(3,))
def _sc_gather(table, idx_main, idx_tail, b):
    _, d = table.shape
    nw, eps_per_w, _ = idx_main.shape
    l = _MAIN + idx_tail.shape[2]
    nrounds = eps_per_w // _NBUF
    nrem = eps_per_w - nrounds * _NBUF
    mesh = plsc.VectorSubcoreMesh(core_axis_name="c", subcore_axis_name="s")

    @functools.partial(
        pl.kernel,
        mesh=mesh,
        out_type=jax.ShapeDtypeStruct((b, l, d), jnp.float32),
        scratch_types=[
            pltpu.VMEM((eps_per_w, _MAIN), jnp.int32),
            pltpu.VMEM((eps_per_w, 2), jnp.int32),
            [pltpu.VMEM((_MAIN, d), jnp.float32) for _ in range(_NBUF)],
            [pltpu.VMEM((2, d), jnp.float32) for _ in range(_NBUF)],
            [pltpu.SemaphoreType.DMA for _ in range(_NBUF)],
            [pltpu.SemaphoreType.DMA for _ in range(_NBUF)],
        ],
    )
    def k(table_hbm, idxm_hbm, idxt_hbm, out_hbm,
          idxm_v, idxt_v, bufs, tbufs, gsems, wsems):
        wid = lax.axis_index("s") * _NC + lax.axis_index("c")
        base = wid * eps_per_w
        pltpu.sync_copy(idxm_hbm.at[wid], idxm_v)
        pltpu.sync_copy(idxt_hbm.at[wid], idxt_v)

        def start_gather(q, e):
            pltpu.async_copy(table_hbm.at[idxm_v.at[e]], bufs[q], gsems[q])
            pltpu.async_copy(table_hbm.at[idxt_v.at[e]], tbufs[q], gsems[q])

        def wait_gather(q, e):
            pltpu.make_async_copy(
                table_hbm.at[idxm_v.at[e]], bufs[q], gsems[q]).wait()
            pltpu.make_async_copy(
                table_hbm.at[idxt_v.at[e]], tbufs[q], gsems[q]).wait()

        def start_write(q, e):
            pltpu.async_copy(
                bufs[q], out_hbm.at[e, pl.ds(0, _MAIN)], wsems[q])
            pltpu.async_copy(
                tbufs[q], out_hbm.at[e, pl.ds(_MAIN, l - _MAIN)], wsems[q])

        def wait_write(q):
            pltpu.make_async_copy(
                bufs[q], out_hbm.at[base, pl.ds(0, _MAIN)], wsems[q]).wait()
            pltpu.make_async_copy(
                tbufs[q], out_hbm.at[base, pl.ds(_MAIN, l - _MAIN)],
                wsems[q]).wait()

        for q in range(_NBUF):
            start_gather(q, q)

        @pl.loop(0, nrounds)
        def _(t):
            c = t * _NBUF
            for q in range(_NBUF):
                wait_gather(q, c + q)
                start_write(q, base + c + q)

            @pl.when(t + 1 < nrounds)
            def _():
                for q in range(_NBUF):
                    wait_write(q)
                    start_gather(q, c + _NBUF + q)

        for q in range(nrem):
            e = nrounds * _NBUF + q
            wait_write(q)
            start_gather(q, e)
        for q in range(nrem):
            e = nrounds * _NBUF + q
            wait_gather(q, e)
            start_write(q, base + e)

        for q in range(_NBUF):
            wait_write(q)

    return k(table, idx_main, idx_tail)


def kernel(timesteps, table):
    b, l, _ = timesteps.shape
    eps_per_w = b // _NW
    idx = timesteps.astype(jnp.int32).reshape(_NW, eps_per_w, l)
    return _sc_gather(table, idx[:, :, :_MAIN], idx[:, :, _MAIN:], b)
